# baseline (speedup 1.0000x reference)
import jax
import jax.numpy as jnp
from jax import lax
from jax.experimental import pallas as pl
from jax.experimental.pallas import tpu as pltpu
from jax.experimental.pallas import tpu_sc as plsc

_V = 1000000
_B = 16384 * 50
_D = 64
_W = 128
_CHUNK = 128
_NW = 32
_CPW = _B // (_NW * _CHUNK)
_NBUF = 5
_LAG = 4


def _ta_body(t_ref, out_ref):
  t = t_ref[...].T
  out_ref[...] = jnp.concatenate([t, t], axis=1)


def _table_rows(table_t):
  return pl.pallas_call(
      _ta_body,
      grid=(pl.cdiv(_V, 512),),
      in_specs=[pl.BlockSpec((_D, 512), lambda i: (0, i))],
      out_specs=pl.BlockSpec((512, _W), lambda i: (i, 0)),
      out_shape=jax.ShapeDtypeStruct((_V, _W), jnp.float32),
  )(table_t)


def _body(table_hbm, idx_hbm, out_hbm, idx_v, bufs, gsem, ssem):
  c = lax.axis_index("c")
  s = lax.axis_index("s")
  wid = s * 2 + c
  row0 = wid * _CPW
  out0 = row0 * _CHUNK

  pltpu.sync_copy(idx_hbm.at[pl.ds(row0, _CPW)], idx_v)

  def fire_gather(j, b):
    pltpu.async_copy(table_hbm.at[idx_v.at[j]], bufs.at[b], gsem.at[b])

  def wait_gather(j, b):
    pltpu.make_async_copy(table_hbm.at[idx_v.at[j]], bufs.at[b],
                          gsem.at[b]).wait()

  def fire_store(i, b):
    pltpu.async_copy(bufs.at[b, :, pl.ds(0, _D)],
                     out_hbm.at[pl.ds(out0 + i * _CHUNK, _CHUNK)], ssem.at[b])

  def wait_store(i, b):
    pltpu.make_async_copy(bufs.at[b, :, pl.ds(0, _D)],
                          out_hbm.at[pl.ds(out0 + i * _CHUNK, _CHUNK)],
                          ssem.at[b]).wait()

  for j in range(_NBUF):
    fire_gather(j, j % _NBUF)
    if j >= _LAG:
      i = j - _LAG
      wait_gather(i, i % _NBUF)
      fire_store(i, i % _NBUF)

  @pl.loop(1, _CPW // _NBUF)
  def _steady(g):
    for b in range(_NBUF):
      j = g * _NBUF + b
      wait_store(j - _NBUF, b)
      fire_gather(j, b)
      i = j - _LAG
      bi = (b - _LAG) % _NBUF
      wait_gather(i, bi)
      fire_store(i, bi)

  for i in range(_CPW - _LAG, _CPW):
    wait_gather(i, i % _NBUF)
    fire_store(i, i % _NBUF)
  for i in range(_CPW - _NBUF, _CPW):
    wait_store(i, i % _NBUF)


def _gather_sc(table_rows, idx2d):
  mesh = plsc.VectorSubcoreMesh(core_axis_name="c", subcore_axis_name="s")
  run = pl.kernel(
      _body,
      out_type=jax.ShapeDtypeStruct((_B, _D), jnp.float32),
      mesh=mesh,
      compiler_params=pltpu.CompilerParams(use_tc_tiling_on_sc=False),
      scratch_types=[
          pltpu.VMEM((_CPW, _CHUNK), jnp.int32),
          pltpu.VMEM((_NBUF, _CHUNK, _W), jnp.float32),
          pltpu.SemaphoreType.DMA((_NBUF,)),
          pltpu.SemaphoreType.DMA((_NBUF,)),
      ],
  )
  return run(table_rows, idx2d)


def _tc_body(lin_ref, out_ref):
  blk = lin_ref[...].reshape(128, 25, 128)
  for q in range(25):
    out_ref[2 * q] = blk[:, q, 0:_D].T
    out_ref[2 * q + 1] = blk[:, q, _D:2 * _D].T


def _to_final(lin):
  pairs = lin.reshape(409600, 128)
  return pl.pallas_call(
      _tc_body,
      grid=(128,),
      in_specs=[pl.BlockSpec((3200, 128), lambda i: (i, 0))],
      out_specs=pl.BlockSpec((50, _D, 128), lambda i: (0, 0, i)),
      out_shape=jax.ShapeDtypeStruct((50, _D, 16384), jnp.float32),
  )(pairs)


def kernel(x, embedding):
  idx2d = x.astype(jnp.int32).reshape(_B // _CHUNK, _CHUNK)
  table_rows = _table_rows(embedding.T)
  lin = _gather_sc(table_rows, idx2d)
  out_t = _to_final(lin)
  return jnp.transpose(out_t, (2, 0, 1))


# speedup vs baseline: 1.4266x; 1.4266x over previous
import jax
import jax.numpy as jnp
from jax import lax
from jax.experimental import pallas as pl
from jax.experimental.pallas import tpu as pltpu
from jax.experimental.pallas import tpu_sc as plsc

_B = 16384 * 50
_D = 64
_CHUNK = 128
_NW = 32
_CPW = _B // (_NW * _CHUNK)
_NBUF = 10
_LAG = 8


def _body(table_hbm, idx_hbm, out_hbm, idx_v, bufs, gsem, ssem):
  c = lax.axis_index("c")
  s = lax.axis_index("s")
  wid = s * 2 + c
  row0 = wid * _CPW
  out0 = row0 * _CHUNK

  pltpu.sync_copy(idx_hbm.at[pl.ds(row0, _CPW)], idx_v)

  def fire_gather(j, b):
    pltpu.async_copy(table_hbm.at[idx_v.at[j]], bufs.at[b], gsem.at[b])

  def wait_gather(j, b):
    pltpu.make_async_copy(table_hbm.at[idx_v.at[j]], bufs.at[b],
                          gsem.at[b]).wait()

  def fire_store(i, b):
    pltpu.async_copy(bufs.at[b], out_hbm.at[pl.ds(out0 + i * _CHUNK, _CHUNK)],
                     ssem.at[b])

  def wait_store(i, b):
    pltpu.make_async_copy(bufs.at[b],
                          out_hbm.at[pl.ds(out0 + i * _CHUNK, _CHUNK)],
                          ssem.at[b]).wait()

  for j in range(_NBUF):
    fire_gather(j, j % _NBUF)
    if j >= _LAG:
      i = j - _LAG
      wait_gather(i, i % _NBUF)
      fire_store(i, i % _NBUF)

  @pl.loop(1, _CPW // _NBUF)
  def _steady(g):
    for b in range(_NBUF):
      j = g * _NBUF + b
      wait_store(j - _NBUF, b)
      fire_gather(j, b)
      i = j - _LAG
      bi = (b - _LAG) % _NBUF
      wait_gather(i, bi)
      fire_store(i, bi)

  for i in range(_CPW - _LAG, _CPW):
    wait_gather(i, i % _NBUF)
    fire_store(i, i % _NBUF)
  for i in range(_CPW - _NBUF, _CPW):
    wait_store(i, i % _NBUF)


@jax.jit
def _gather(table, idx2d):
  mesh = plsc.VectorSubcoreMesh(core_axis_name="c", subcore_axis_name="s")
  run = pl.kernel(
      _body,
      out_type=jax.ShapeDtypeStruct((_B, _D), jnp.float32),
      mesh=mesh,
      compiler_params=pltpu.CompilerParams(use_tc_tiling_on_sc=False),
      scratch_types=[
          pltpu.VMEM((_CPW, _CHUNK), jnp.int32),
          pltpu.VMEM((_NBUF, _CHUNK, _D), jnp.float32),
          pltpu.SemaphoreType.DMA((_NBUF,)),
          pltpu.SemaphoreType.DMA((_NBUF,)),
      ],
  )
  return run(table, idx2d)


def kernel(x, embedding):
  idx2d = x.astype(jnp.int32).reshape(_B // _CHUNK, _CHUNK)
  out = _gather(embedding, idx2d)
  return out.reshape(x.shape[0], x.shape[1], _D)
